# baseline (speedup 1.0000x reference)
import jax
import jax.numpy as jnp
from jax.experimental import pallas as pl
from jax.experimental.pallas import tpu as pltpu

_BSZ = 1024
_BASE = 512 * 128
_BI = 128
_BJ = 8192
_GI = _BSZ // _BI
_GJ = _BASE // _BJ
_K = _BSZ // 2


def _fused_kernel(step_ref, inp_ref, tgt_ref, out_ref, err_ref):
    i = pl.program_id(0)
    j = pl.program_id(1)
    diff = inp_ref[...] - tgt_ref[...]
    part = jnp.sum(diff * diff, axis=1, keepdims=True).reshape(1, _BI) * (
        1.0 / _BASE)

    @pl.when(j == 0)
    def _init():
        err_ref[0:1, pl.ds(i * _BI, _BI)] = part

    @pl.when(j > 0)
    def _acc():
        err_ref[0:1, pl.ds(i * _BI, _BI)] += part

    @pl.when((i == _GI - 1) & (j == _GJ - 1))
    def _finalize():
        errors = err_ref[0, :]
        U = jnp.mean(errors)
        var = jnp.sum((errors - U) ** 2) * (1.0 / (_BSZ - 1))
        Sigma = jnp.sqrt(var) + 1e-6
        u = 0.1 * U
        sigma = 0.1 * Sigma + 0.9
        z = jnp.abs(errors - u) * (1.0 / sigma)
        nz = -z
        e = jnp.exp(nz - jnp.max(nz))
        w1 = e * (1.0 / jnp.sum(e))
        w1 = w1 * (1.0 / jnp.mean(w1))
        col = w1.reshape(_BSZ, 1)
        row = w1.reshape(1, _BSZ)
        lt = (row < col).astype(jnp.float32)
        ji = jax.lax.broadcasted_iota(jnp.int32, (_BSZ, _BSZ), 1)
        ii = jax.lax.broadcasted_iota(jnp.int32, (_BSZ, _BSZ), 0)
        tie = ((row == col) & (ji < ii)).astype(jnp.float32)
        rank = jnp.sum(lt + tie, axis=1)
        w1m = jnp.where(rank >= _K, w1, 0.0)
        step = step_ref[0, 0]
        w = (1.0 + (step - 1.0) * w1m) * (1.0 / step)
        out_ref[...] = jnp.mean(errors * w).reshape(1, 1)


def kernel(input, target, global_step):
    step = jnp.asarray(global_step, dtype=input.dtype).reshape(1, 1)
    inp2 = input.reshape(_BSZ, _BASE)
    tgt2 = target.reshape(_BSZ, _BASE)
    out = pl.pallas_call(
        _fused_kernel,
        grid=(_GI, _GJ),
        in_specs=[
            pl.BlockSpec((1, 1), lambda i, j: (0, 0)),
            pl.BlockSpec((_BI, _BJ), lambda i, j: (i, j)),
            pl.BlockSpec((_BI, _BJ), lambda i, j: (i, j)),
        ],
        out_specs=pl.BlockSpec((1, 1), lambda i, j: (0, 0)),
        out_shape=jax.ShapeDtypeStruct((1, 1), jnp.float32),
        scratch_shapes=[pltpu.VMEM((1, _BSZ), jnp.float32)],
        compiler_params=pltpu.CompilerParams(
            dimension_semantics=("arbitrary", "arbitrary"),
        ),
    )(step, inp2, tgt2)
    return out[0, 0]


# speedup vs baseline: 1.0000x; 1.0000x over previous
import jax
import jax.numpy as jnp
from jax.experimental import pallas as pl
from jax.experimental.pallas import tpu as pltpu

_BSZ = 1024
_BASE = 512 * 128
_BI = 128
_BJ = 8192
_GI = _BSZ // _BI
_GJ = _BASE // _BJ
_K = _BSZ // 2


def _err_kernel(inp_ref, tgt_ref, err_ref):
    j = pl.program_id(1)
    diff = inp_ref[...] - tgt_ref[...]
    part = jnp.sum(diff * diff, axis=1).reshape(1, _BI) * (1.0 / _BASE)

    @pl.when(j == 0)
    def _init():
        err_ref[...] = part

    @pl.when(j > 0)
    def _acc():
        err_ref[...] += part


def _loss_kernel(step_ref, err_ref, out_ref):
    errors = err_ref[0, :]
    U = jnp.mean(errors)
    var = jnp.sum((errors - U) ** 2) * (1.0 / (_BSZ - 1))
    Sigma = jnp.sqrt(var) + 1e-6
    u = 0.1 * U
    sigma = 0.1 * Sigma + 0.9
    z = jnp.abs(errors - u) * (1.0 / sigma)
    nz = -z
    e = jnp.exp(nz - jnp.max(nz))
    w1 = e * (1.0 / jnp.sum(e))
    w1 = w1 * (1.0 / jnp.mean(w1))
    col = w1.reshape(_BSZ, 1)
    row = w1.reshape(1, _BSZ)
    lt = (row < col).astype(jnp.float32)
    ji = jax.lax.broadcasted_iota(jnp.int32, (_BSZ, _BSZ), 1)
    ii = jax.lax.broadcasted_iota(jnp.int32, (_BSZ, _BSZ), 0)
    tie = ((row == col) & (ji < ii)).astype(jnp.float32)
    rank = jnp.sum(lt + tie, axis=1)
    w1m = jnp.where(rank >= _K, w1, 0.0)
    step = step_ref[0, 0]
    w = (1.0 + (step - 1.0) * w1m) * (1.0 / step)
    out_ref[...] = jnp.mean(errors * w).reshape(1, 1)


def kernel(input, target, global_step):
    step = jnp.asarray(global_step, dtype=input.dtype).reshape(1, 1)
    inp2 = input.reshape(_BSZ, _BASE)
    tgt2 = target.reshape(_BSZ, _BASE)
    errors = pl.pallas_call(
        _err_kernel,
        grid=(_GI, _GJ),
        in_specs=[
            pl.BlockSpec((_BI, _BJ), lambda i, j: (i, j)),
            pl.BlockSpec((_BI, _BJ), lambda i, j: (i, j)),
        ],
        out_specs=pl.BlockSpec((1, _BI), lambda i, j: (0, i)),
        out_shape=jax.ShapeDtypeStruct((1, _BSZ), jnp.float32),
        compiler_params=pltpu.CompilerParams(
            dimension_semantics=("parallel", "arbitrary"),
        ),
    )(inp2, tgt2)
    out = pl.pallas_call(
        _loss_kernel,
        in_specs=[
            pl.BlockSpec((1, 1), lambda: (0, 0)),
            pl.BlockSpec((1, _BSZ), lambda: (0, 0)),
        ],
        out_specs=pl.BlockSpec((1, 1), lambda: (0, 0)),
        out_shape=jax.ShapeDtypeStruct((1, 1), jnp.float32),
    )(step, errors)
    return out[0, 0]
